# Initial kernel scaffold; baseline (speedup 1.0000x reference)
import jax
import jax.numpy as jnp
from jax.experimental import pallas as pl


def kernel(seq_len, relative_attention_bias):
    raise NotImplementedError("write your pallas kernel here")


# speedup vs baseline: 9.8353x; 9.8353x over previous
import functools

import jax
import jax.numpy as jnp
from jax import lax
from jax.experimental import pallas as pl
from jax.experimental.pallas import tpu as pltpu
from jax.experimental.pallas import tpu_sc as plsc

S = 2048
H = 16
MAX_REL = 32
NUM_TABLE_ROWS = 2 * MAX_REL + 1

NC = 2
NS = 16
NW = NC * NS
RPW = S // NW

WIN_ROWS = (RPW - 1) + S
WIN_LEN = WIN_ROWS * H


def _sc_body(table_hbm, out_hbm, table_v, win_v):
    wid = lax.axis_index("s") * NC + lax.axis_index("c")
    base = wid * RPW

    pltpu.sync_copy(table_hbm, table_v)

    def build(tp, carry):
        idx = jnp.clip((RPW - 1) + base - tp, -MAX_REL, MAX_REL) + MAX_REL
        win_v[tp, :] = table_v[idx, :]
        return carry

    lax.fori_loop(0, WIN_ROWS, build, 0, unroll=4)

    def emit(r, carry):
        start = pl.multiple_of(RPW - 1 - r, 1)
        pltpu.sync_copy(win_v.at[pl.ds(start, S), :], out_hbm.at[base + r])
        return carry

    lax.fori_loop(0, RPW, emit, 0)


def kernel(seq_len, relative_attention_bias):
    mesh = plsc.VectorSubcoreMesh(core_axis_name="c", subcore_axis_name="s")
    out = pl.kernel(
        _sc_body,
        mesh=mesh,
        out_type=jax.ShapeDtypeStruct((S, S, H), jnp.float32),
        scratch_types=[
            pltpu.VMEM((NUM_TABLE_ROWS, H), jnp.float32),
            pltpu.VMEM((WIN_ROWS, H), jnp.float32),
        ],
        compiler_params=pltpu.CompilerParams(use_tc_tiling_on_sc=False),
    )(relative_attention_bias)
    return out


# speedup vs baseline: 9.8390x; 1.0004x over previous
import functools

import jax
import jax.numpy as jnp
from jax import lax
from jax.experimental import pallas as pl
from jax.experimental.pallas import tpu as pltpu
from jax.experimental.pallas import tpu_sc as plsc

S = 2048
H = 16
MAX_REL = 32
NUM_TABLE_ROWS = 2 * MAX_REL + 1

NC = 2
NS = 16
NW = NC * NS
RPW = S // NW

WIN_ROWS = (RPW - 1) + S
WIN_LEN = WIN_ROWS * H


PIPE = 8


def _sc_body(table_hbm, out_hbm, table_v, win_v, sem):
    wid = lax.axis_index("s") * NC + lax.axis_index("c")
    base = wid * RPW

    pltpu.sync_copy(table_hbm, table_v)

    def build(tp, carry):
        idx = jnp.clip((RPW - 1) + base - tp, -MAX_REL, MAX_REL) + MAX_REL
        win_v[tp, :] = table_v[idx, :]
        return carry

    lax.fori_loop(0, WIN_ROWS, build, 0, unroll=4)

    def emit(r, carry):
        start = RPW - 1 - r
        pltpu.make_async_copy(
            win_v.at[pl.ds(start, S), :], out_hbm.at[base + r], sem
        ).start()

        @pl.when(r >= PIPE)
        def _drain_one():
            pltpu.make_async_copy(
                win_v.at[pl.ds(0, S), :], out_hbm.at[base], sem
            ).wait()

        return carry

    lax.fori_loop(0, RPW, emit, 0)

    def drain(r, carry):
        pltpu.make_async_copy(
            win_v.at[pl.ds(0, S), :], out_hbm.at[base], sem
        ).wait()
        return carry

    lax.fori_loop(0, PIPE, drain, 0)


def kernel(seq_len, relative_attention_bias):
    mesh = plsc.VectorSubcoreMesh(core_axis_name="c", subcore_axis_name="s")
    out = pl.kernel(
        _sc_body,
        mesh=mesh,
        out_type=jax.ShapeDtypeStruct((S, S, H), jnp.float32),
        scratch_types=[
            pltpu.VMEM((NUM_TABLE_ROWS, H), jnp.float32),
            pltpu.VMEM((WIN_ROWS, H), jnp.float32),
            pltpu.SemaphoreType.DMA,
        ],
        compiler_params=pltpu.CompilerParams(use_tc_tiling_on_sc=False),
    )(relative_attention_bias)
    return out
